# Initial kernel scaffold; baseline (speedup 1.0000x reference)
import jax
import jax.numpy as jnp
from jax.experimental import pallas as pl


def kernel(x, x_train, y_train):
    raise NotImplementedError("write your pallas kernel here")


# speedup vs baseline: 4.3962x; 4.3962x over previous
import functools

import jax
import jax.numpy as jnp
from jax.experimental import pallas as pl
from jax.experimental.pallas import tpu as pltpu

NUM_CLASSES = 1000
TOP_K = 200
INV_T = 0.1
CHUNK = 2048
ROWS = 32
CPAD = 1024
VOTE_DTYPE = jnp.float32

import numpy as np

_I32_MIN = np.int32(-(2**31))
_FLIP = np.int32(0x7FFFFFFF)


def _f32_keys(sim):
    bits = jax.lax.bitcast_convert_type(sim, jnp.int32)
    return jnp.where(bits < 0, bits ^ _FLIP, bits)


def _keys_to_f32(keys):
    bits = jnp.where(keys < 0, keys ^ _FLIP, keys)
    return jax.lax.bitcast_convert_type(bits, jnp.float32)


def _knn_body(n_train, n_chunks, x_ref, xt_ref, y_ref, out_ref, keys_ref):
    c = pl.program_id(1)
    sim = jax.lax.dot_general(
        x_ref[...].astype(jnp.bfloat16), xt_ref[...].astype(jnp.bfloat16),
        (((1,), (1,)), ((), ())),
        preferred_element_type=jnp.float32)
    keys = _f32_keys(sim)
    col = c * CHUNK + jax.lax.broadcasted_iota(jnp.int32, keys.shape, 1)
    keys = jnp.where(col < n_train, keys, _I32_MIN)
    keys_ref[:, pl.ds(c * CHUNK, CHUNK)] = keys

    @pl.when(c == n_chunks - 1)
    def _finalize():
        rows = keys_ref.shape[0]
        all_keys = keys_ref[...]

        cnt_pos = jnp.sum((all_keys >= 0).astype(jnp.int32), axis=1,
                          keepdims=True)
        thr0 = jnp.where(cnt_pos >= TOP_K, jnp.int32(0), _I32_MIN)

        def bit_step(i, thr):
            cand = thr | jnp.left_shift(jnp.int32(1), 30 - i)
            cnt = jnp.sum((all_keys >= cand).astype(jnp.int32), axis=1,
                          keepdims=True)
            return jnp.where(cnt >= TOP_K, cand, thr)

        thr = jax.lax.fori_loop(0, 31, bit_step, thr0)

        cls_iota = jax.lax.broadcasted_iota(jnp.int32, (CPAD, CHUNK), 0)

        def chunk_step(t, acc):
            kch = keys_ref[:, pl.ds(t * CHUNK, CHUNK)]
            simc = _keys_to_f32(kch)
            w = jnp.where(kch >= thr, jnp.exp(simc * INV_T), 0.0)
            ych = y_ref[t]
            onehot_t = (ych == cls_iota).astype(VOTE_DTYPE)
            return acc + jax.lax.dot_general(
                w.astype(VOTE_DTYPE), onehot_t, (((1,), (1,)), ((), ())),
                precision=jax.lax.Precision.HIGHEST,
                preferred_element_type=jnp.float32)

        scores = jax.lax.fori_loop(
            0, n_chunks, chunk_step, jnp.zeros((rows, CPAD), jnp.float32))

        pad_iota = jax.lax.broadcasted_iota(jnp.int32, (rows, CPAD), 1)
        scores = jnp.where(pad_iota < NUM_CLASSES, scores, -1.0)

        def shifted(a, j):
            left = jnp.concatenate([a[:, j:], a[:, :j]], axis=1)
            right = jnp.concatenate([a[:, -j:], a[:, :-j]], axis=1)
            return left, right

        v = scores
        ind = pad_iota
        size = 2
        while size <= CPAD:
            j = size // 2
            while j >= 1:
                is_low = (pad_iota & j) == 0
                desc = (pad_iota & size) == 0
                vl, vr = shifted(v, j)
                il, ir = shifted(ind, j)
                pv = jnp.where(is_low, vl, vr)
                pi = jnp.where(is_low, il, ir)
                lo_v = jnp.where(is_low, v, pv)
                hi_v = jnp.where(is_low, pv, v)
                lo_i = jnp.where(is_low, ind, pi)
                hi_i = jnp.where(is_low, pi, ind)
                lo_less = (lo_v < hi_v) | ((lo_v == hi_v) & (lo_i > hi_i))
                swap = lo_less == desc
                v = jnp.where(is_low, jnp.where(swap, hi_v, lo_v),
                              jnp.where(swap, lo_v, hi_v))
                ind = jnp.where(is_low, jnp.where(swap, hi_i, lo_i),
                                jnp.where(swap, lo_i, hi_i))
                j //= 2
            size *= 2

        out_ref[...] = ind[:, :NUM_CLASSES]


@jax.jit
def kernel(x, x_train, y_train):
    batch, feat = x.shape
    n_train = x_train.shape[0]
    n_pad = ((n_train + CHUNK - 1) // CHUNK) * CHUNK
    n_chunks = n_pad // CHUNK
    xt = jnp.pad(x_train, ((0, n_pad - n_train), (0, 0)))
    y2 = jnp.pad(y_train.astype(jnp.int32), (0, n_pad - n_train))
    y2 = y2.reshape(n_chunks, 1, CHUNK)

    body = functools.partial(_knn_body, n_train, n_chunks)
    return pl.pallas_call(
        body,
        grid=(batch // ROWS, n_chunks),
        in_specs=[
            pl.BlockSpec((ROWS, feat), lambda r, c: (r, 0)),
            pl.BlockSpec((CHUNK, feat), lambda r, c: (c, 0)),
            pl.BlockSpec((n_chunks, 1, CHUNK), lambda r, c: (0, 0, 0)),
        ],
        out_specs=pl.BlockSpec((ROWS, NUM_CLASSES), lambda r, c: (r, 0)),
        out_shape=jax.ShapeDtypeStruct((batch, NUM_CLASSES), jnp.int32),
        scratch_shapes=[pltpu.VMEM((ROWS, n_pad), jnp.int32)],
    )(x, xt, y2)
